# baseline (speedup 1.0000x reference)
import functools
import jax
import jax.numpy as jnp
from jax import lax
from jax.experimental import pallas as pl
from jax.experimental.pallas import tpu as pltpu
from jax.experimental.pallas import tpu_sc as plsc

F32 = jnp.float32
HIGH = lax.Precision.HIGHEST

N = 10000
E = 320000
F = 128
H1, U1 = 8, 8
OUT = 7

NC, NS, L = 2, 16, 16
NW = NC * NS
EW = E // NW
CH = 80
NCHUNK = EW // CH
NPAD = 10240
RPT = NPAD // NS

BLK = 400
NBLK = N // BLK
BLK2 = 512
NBLK2 = NPAD // BLK2


def _dot(a, b):
    return jnp.dot(a, b, preferred_element_type=F32, precision=HIGH)


def _k1_body(x_ref, ws_ref, wd_ref, ts_ref, td_ref):
    x = x_ref[...]
    ts_ref[...] = _dot(x, ws_ref[...])
    td_ref[...] = _dot(x, wd_ref[...])


def _k1(x, wsrc, wdst):
    return pl.pallas_call(
        _k1_body,
        grid=(NBLK,),
        in_specs=[
            pl.BlockSpec((BLK, F), lambda i: (i, 0)),
            pl.BlockSpec((F, 80), lambda i: (0, 0)),
            pl.BlockSpec((F, 16), lambda i: (0, 0)),
        ],
        out_specs=[
            pl.BlockSpec((BLK, 80), lambda i: (i, 0)),
            pl.BlockSpec((BLK, 16), lambda i: (i, 0)),
        ],
        out_shape=[
            jax.ShapeDtypeStruct((N, 80), F32),
            jax.ShapeDtypeStruct((N, 16), F32),
        ],
    )(x, wsrc, wdst)


def _k3_body(a0_ref, a1_ref, r_ref, wt_ref, wd_ref, ts_ref, td_ref):
    s = a0_ref[...] + a1_ref[...]
    numer = s[:, 0:64]
    den = s[:, 64:72]
    rrep = _dot(1.0 / (den + 1e-9), r_ref[...])
    x2 = jnp.maximum(numer, 0.0) * rrep
    t = _dot(x2, wt_ref[...])
    col = lax.broadcasted_iota(jnp.int32, t.shape, 1)
    ts_ref[...] = jnp.where(col == 7, 1.0, t)
    td_ref[...] = _dot(x2, wd_ref[...])


def _k3(acc1, rmat, wt, wd2):
    return pl.pallas_call(
        _k3_body,
        grid=(NBLK2,),
        in_specs=[
            pl.BlockSpec((BLK2, 80), lambda i: (i, 0)),
            pl.BlockSpec((BLK2, 80), lambda i: (i + NBLK2, 0)),
            pl.BlockSpec((H1, 64), lambda i: (0, 0)),
            pl.BlockSpec((64, 16), lambda i: (0, 0)),
            pl.BlockSpec((64, 16), lambda i: (0, 0)),
        ],
        out_specs=[
            pl.BlockSpec((BLK2, 16), lambda i: (i, 0)),
            pl.BlockSpec((BLK2, 16), lambda i: (i, 0)),
        ],
        out_shape=[
            jax.ShapeDtypeStruct((NPAD, 16), F32),
            jax.ShapeDtypeStruct((NPAD, 16), F32),
        ],
    )(acc1, acc1, rmat, wt, wd2)


def _k5_body(a0_ref, a1_ref, o_ref):
    s = a0_ref[...] + a1_ref[...]
    numer = s[:, 0:OUT]
    den = s[:, OUT:OUT + 1]
    o_ref[...] = jnp.maximum(numer, 0.0) / (den + 1e-9)


def _k5(acc2):
    return pl.pallas_call(
        _k5_body,
        grid=(NBLK2,),
        in_specs=[
            pl.BlockSpec((BLK2, 16), lambda i: (i, 0)),
            pl.BlockSpec((BLK2, 16), lambda i: (i + NBLK2, 0)),
        ],
        out_specs=pl.BlockSpec((BLK2, OUT), lambda i: (i, 0)),
        out_shape=jax.ShapeDtypeStruct((NPAD, OUT), F32),
    )(acc2, acc2)


def _sc_mesh():
    return plsc.VectorSubcoreMesh(core_axis_name="c", subcore_axis_name="s")


def _zero_acc(zbuf, acc, width, s):
    zero = jnp.zeros((L,), F32)
    nvec = width // L

    def zrow(i, _):
        for j in range(nvec):
            zbuf[i, pl.ds(j * L, L)] = zero
        return 0

    lax.fori_loop(0, 128, zrow, 0)
    for k in range(5):
        pltpu.sync_copy(zbuf, acc.at[pl.ds(s * RPT + k * 128, 128)])


def _sc_layer1(tsrc, tdst, src_ix, dst_ix):
    return _sc_edge_kernel(tsrc, tdst, src_ix, dst_ix, 80, _edges1)


def _edges1(srows, drows, vals):
    giota = lax.iota(jnp.int32, L)
    mask8 = giota < 8
    half = jnp.where(mask8, 0, 1)

    @plsc.parallel_loop(0, CH, unroll=8)
    def edge(e):
        t = srows[e, pl.ds(64, L)] + drows[e, :]
        t = jnp.maximum(t, 0.2 * t)
        sc = jnp.exp(t)
        vals[e, pl.ds(64, L)] = jnp.where(mask8, sc, 0.0)
        for j in range(4):
            b = jnp.take(sc, 2 * j + half)
            vals[e, pl.ds(16 * j, L)] = b * srows[e, pl.ds(16 * j, L)]


def _sc_edge_kernel(tsrc, tdst, src_ix, dst_ix, width, edge_fn):
    @functools.partial(
        pl.kernel,
        out_type=jax.ShapeDtypeStruct((2 * NPAD, width), F32),
        mesh=_sc_mesh(),
        scratch_types=[
            pltpu.VMEM((EW,), jnp.int32),
            pltpu.VMEM((EW,), jnp.int32),
            pltpu.VMEM((2, CH, width), F32),
            pltpu.VMEM((2, CH, 16), F32),
            pltpu.VMEM((2, CH, width), F32),
            pltpu.VMEM((CH,), jnp.int32),
            pltpu.VMEM((CH,), jnp.int32),
            pltpu.VMEM((128, width), F32),
            pltpu.VMEM_SHARED((NPAD, width), F32),
            pltpu.SemaphoreType.DMA,
            pltpu.SemaphoreType.DMA,
            pltpu.SemaphoreType.DMA,
            pltpu.SemaphoreType.DMA,
        ],
        compiler_params=pltpu.CompilerParams(use_tc_tiling_on_sc=False),
    )
    def k(tsrc_h, tdst_h, six_h, dix_h, out_h,
          sixall, dixall, srows2, drows2, vals2, sdix0, sdix1, zbuf, acc,
          gsem0, gsem1, ssem0, ssem1):
        c = lax.axis_index("c")
        s = lax.axis_index("s")
        w = s * NC + c

        _zero_acc(zbuf, acc, width, s)

        ebase = pl.multiple_of(w * EW, 8)
        pltpu.sync_copy(six_h.at[pl.ds(ebase, EW)], sixall)
        pltpu.sync_copy(dix_h.at[pl.ds(ebase, EW)], dixall)
        plsc.subcore_barrier()

        bufs = [
            (srows2.at[0], drows2.at[0], vals2.at[0], sdix0, gsem0, ssem0),
            (srows2.at[1], drows2.at[1], vals2.at[1], sdix1, gsem1, ssem1),
        ]

        def start_gather(ci, b):
            srows, drows, _, _, gsem, _ = bufs[b]
            off = ci * CH
            pltpu.async_copy(tsrc_h.at[sixall.at[pl.ds(off, CH)]], srows, gsem)
            pltpu.async_copy(tdst_h.at[dixall.at[pl.ds(off, CH)]], drows, gsem)

        def wait_gather(b):
            srows, drows, _, _, gsem, _ = bufs[b]
            pltpu.make_async_copy(tsrc_h.at[sixall.at[pl.ds(0, CH)]], srows, gsem).wait()
            pltpu.make_async_copy(tdst_h.at[dixall.at[pl.ds(0, CH)]], drows, gsem).wait()

        def wait_scatter(b):
            _, _, vals, sdix, _, ssem = bufs[b]
            pltpu.make_async_copy(vals, acc.at[sdix], ssem).wait()

        def process(ci, b, first):
            srows, drows, vals, sdix, _, ssem = bufs[b]
            wait_gather(b)
            if first:
                pl.when(ci >= 2)(lambda: wait_scatter(b))
            else:
                wait_scatter(b)
            off = ci * CH
            for j in range(CH // L):
                sdix[pl.ds(j * L, L)] = dixall[pl.ds(off + j * L, L)]
            edge_fn(srows, drows, vals)
            pltpu.async_copy(vals, acc.at[sdix], ssem, add=True)

        start_gather(0, 0)

        def pair(i, _):
            start_gather(2 * i + 1, 1)
            process(2 * i, 0, True)
            start_gather(2 * i + 2, 0)
            process(2 * i + 1, 1, True)
            return 0

        lax.fori_loop(0, (NCHUNK - 1) // 2, pair, 0)
        process(NCHUNK - 1, 0, False)
        wait_scatter(1)
        wait_scatter(0)

        plsc.subcore_barrier()
        pltpu.sync_copy(acc.at[pl.ds(s * RPT, RPT)],
                        out_h.at[pl.ds(c * NPAD + s * RPT, RPT)])

    return k(tsrc, tdst, src_ix, dst_ix)


def _sc_layer2(tsrc, sd1d, ss1d, src_ix, dst_ix):
    @functools.partial(
        pl.kernel,
        out_type=jax.ShapeDtypeStruct((2 * NPAD, 16), F32),
        mesh=_sc_mesh(),
        scratch_types=[
            pltpu.VMEM((EW,), jnp.int32),
            pltpu.VMEM((EW,), jnp.int32),
            pltpu.VMEM((2, CH, 16), F32),
            pltpu.VMEM((2, CH, 16), F32),
            pltpu.VMEM((CH,), jnp.int32),
            pltpu.VMEM((CH,), jnp.int32),
            pltpu.VMEM((2, CH), F32),
            pltpu.VMEM((2, CH), F32),
            pltpu.VMEM((128, 16), F32),
            pltpu.VMEM_SHARED((NPAD, 16), F32),
            pltpu.SemaphoreType.DMA,
            pltpu.SemaphoreType.DMA,
            pltpu.SemaphoreType.DMA,
            pltpu.SemaphoreType.DMA,
        ],
        compiler_params=pltpu.CompilerParams(use_tc_tiling_on_sc=False),
    )
    def k(tsrc_h, sd_h, ss_h, six_h, dix_h, out_h,
          sixall, dixall, srows2, vals2, sdix0, sdix1, sdch2, ssch2, zbuf,
          acc, gsem0, gsem1, ssem0, ssem1):
        c = lax.axis_index("c")
        s = lax.axis_index("s")
        w = s * NC + c

        _zero_acc(zbuf, acc, 16, s)

        ebase = pl.multiple_of(w * EW, 8)
        pltpu.sync_copy(six_h.at[pl.ds(ebase, EW)], sixall)
        pltpu.sync_copy(dix_h.at[pl.ds(ebase, EW)], dixall)
        plsc.subcore_barrier()

        giota = lax.iota(jnp.int32, L)
        mask8 = giota < 8
        lane_consts = [jnp.full((L,), e, jnp.int32) for e in range(L)]
        bufs = [
            (srows2.at[0], vals2.at[0], sdix0, sdch2.at[0], ssch2.at[0], gsem0, ssem0),
            (srows2.at[1], vals2.at[1], sdix1, sdch2.at[1], ssch2.at[1], gsem1, ssem1),
        ]

        def start_gather(ci, b):
            srows, _, _, sdch, ssch, gsem, _ = bufs[b]
            off = ci * CH
            pltpu.async_copy(tsrc_h.at[sixall.at[pl.ds(off, CH)]], srows, gsem)
            pltpu.async_copy(sd_h.at[dixall.at[pl.ds(off, CH)]], sdch, gsem)
            pltpu.async_copy(ss_h.at[sixall.at[pl.ds(off, CH)]], ssch, gsem)

        def wait_gather(b):
            srows, _, _, sdch, ssch, gsem, _ = bufs[b]
            pltpu.make_async_copy(tsrc_h.at[sixall.at[pl.ds(0, CH)]], srows, gsem).wait()
            pltpu.make_async_copy(sd_h.at[dixall.at[pl.ds(0, CH)]], sdch, gsem).wait()
            pltpu.make_async_copy(ss_h.at[sixall.at[pl.ds(0, CH)]], ssch, gsem).wait()

        def wait_scatter(b):
            _, vals, sdix, _, _, _, ssem = bufs[b]
            pltpu.make_async_copy(vals, acc.at[sdix], ssem).wait()

        def process(ci, b, guard):
            srows, vals, sdix, sdch, ssch, _, ssem = bufs[b]
            wait_gather(b)
            if guard:
                pl.when(ci >= 2)(lambda: wait_scatter(b))
            else:
                wait_scatter(b)
            off = ci * CH
            for j in range(CH // L):
                sdix[pl.ds(j * L, L)] = dixall[pl.ds(off + j * L, L)]
            @plsc.parallel_loop(0, CH // L, unroll=5)
            def group(kk):
                t = sdch[pl.ds(kk * L, L)] + ssch[pl.ds(kk * L, L)]
                t = jnp.maximum(t, 0.2 * t)
                sc16 = jnp.exp(t)
                for e in range(L):
                    b_ = jnp.take(sc16, lane_consts[e])
                    vals[kk * L + e, :] = jnp.where(mask8, b_ * srows[kk * L + e, :], 0.0)
            pltpu.async_copy(vals, acc.at[sdix], ssem, add=True)

        start_gather(0, 0)

        def pair(i, _):
            start_gather(2 * i + 1, 1)
            process(2 * i, 0, True)
            start_gather(2 * i + 2, 0)
            process(2 * i + 1, 1, True)
            return 0

        lax.fori_loop(0, (NCHUNK - 1) // 2, pair, 0)
        process(NCHUNK - 1, 0, False)
        wait_scatter(1)
        wait_scatter(0)

        plsc.subcore_barrier()
        pltpu.sync_copy(acc.at[pl.ds(s * RPT, RPT)],
                        out_h.at[pl.ds(c * NPAD + s * RPT, RPT)])

    return k(tsrc, sd1d, ss1d, src_ix, dst_ix)


def kernel(node_states, edges, W1, A1, W2, A2, training=False):
    del training
    x = node_states.astype(F32)

    wc = W1.transpose(1, 0, 2).reshape(F, H1 * U1)
    eye = jnp.eye(H1, dtype=F32)
    a_dst = A1[:, :U1, 0]
    a_src = A1[:, U1:, 0]
    ad_mat = (a_dst[:, :, None] * eye[:, None, :]).reshape(H1 * U1, H1)
    as_mat = (a_src[:, :, None] * eye[:, None, :]).reshape(H1 * U1, H1)
    z8 = jnp.zeros((F, 8), F32)
    wsrc = jnp.concatenate([wc, wc @ as_mat, z8], axis=1)
    wdst = jnp.concatenate([wc @ ad_mat, z8], axis=1)

    rmat = (eye[:, :, None] * jnp.ones((1, 1, U1), F32)).reshape(H1, H1 * U1)

    w2 = W2[0]
    a2d = A2[0, :OUT, 0]
    a2s = A2[0, OUT:, 0]
    z1 = jnp.zeros((H1 * U1, 1), F32)
    z7 = jnp.zeros((H1 * U1, 7), F32)
    wt = jnp.concatenate([w2, z1, (w2 @ a2s)[:, None], z7], axis=1)
    wd2 = jnp.concatenate([jnp.zeros((H1 * U1, 8), F32),
                           (w2 @ a2d)[:, None], z7], axis=1)

    src = edges[:, 1].astype(jnp.int32)
    dst = edges[:, 0].astype(jnp.int32)

    tsrc1, tdst1 = _k1(x, wsrc, wdst)
    acc1 = _sc_layer1(tsrc1, tdst1, src, dst)
    tsrc2, tdst2 = _k3(acc1, rmat, wt, wd2)
    sd1d = tdst2[:, 8]
    ss1d = tsrc2[:, 8]
    acc2 = _sc_layer2(tsrc2, sd1d, ss1d, src, dst)
    return _k5(acc2)[:N]


# speedup vs baseline: 1.0002x; 1.0002x over previous
import functools
import jax
import jax.numpy as jnp
from jax import lax
from jax.experimental import pallas as pl
from jax.experimental.pallas import tpu as pltpu
from jax.experimental.pallas import tpu_sc as plsc

F32 = jnp.float32
HIGH = lax.Precision.HIGHEST

N = 10000
E = 320000
F = 128
H1, U1 = 8, 8
OUT = 7

NC, NS, L = 2, 16, 16
NW = NC * NS
EW = E // NW
CH = 80
NCHUNK = EW // CH
NPAD = 10240
RPT = NPAD // NS

BLK = 400
NBLK = N // BLK
BLK2 = 512
NBLK2 = NPAD // BLK2


def _dot(a, b):
    return jnp.dot(a, b, preferred_element_type=F32, precision=HIGH)


def _k1_body(x_ref, ws_ref, wd_ref, ts_ref, td_ref):
    x = x_ref[...]
    ts_ref[...] = _dot(x, ws_ref[...])
    td_ref[...] = _dot(x, wd_ref[...])


def _k1(x, wsrc, wdst):
    return pl.pallas_call(
        _k1_body,
        grid=(NBLK,),
        in_specs=[
            pl.BlockSpec((BLK, F), lambda i: (i, 0)),
            pl.BlockSpec((F, 80), lambda i: (0, 0)),
            pl.BlockSpec((F, 16), lambda i: (0, 0)),
        ],
        out_specs=[
            pl.BlockSpec((BLK, 80), lambda i: (i, 0)),
            pl.BlockSpec((BLK, 16), lambda i: (i, 0)),
        ],
        out_shape=[
            jax.ShapeDtypeStruct((N, 80), F32),
            jax.ShapeDtypeStruct((N, 16), F32),
        ],
    )(x, wsrc, wdst)


def _k3_body(a0_ref, a1_ref, r_ref, wt_ref, wd_ref, ts_ref, td_ref):
    s = a0_ref[...] + a1_ref[...]
    numer = s[:, 0:64]
    den = s[:, 64:72]
    rrep = _dot(1.0 / (den + 1e-9), r_ref[...])
    x2 = jnp.maximum(numer, 0.0) * rrep
    t = _dot(x2, wt_ref[...])
    col = lax.broadcasted_iota(jnp.int32, t.shape, 1)
    ts_ref[...] = jnp.where(col == 7, 1.0, t)
    td_ref[...] = _dot(x2, wd_ref[...])


def _k3(acc1, rmat, wt, wd2):
    return pl.pallas_call(
        _k3_body,
        grid=(NBLK2,),
        in_specs=[
            pl.BlockSpec((BLK2, 80), lambda i: (i, 0)),
            pl.BlockSpec((BLK2, 80), lambda i: (i + NBLK2, 0)),
            pl.BlockSpec((H1, 64), lambda i: (0, 0)),
            pl.BlockSpec((64, 16), lambda i: (0, 0)),
            pl.BlockSpec((64, 16), lambda i: (0, 0)),
        ],
        out_specs=[
            pl.BlockSpec((BLK2, 16), lambda i: (i, 0)),
            pl.BlockSpec((BLK2, 16), lambda i: (i, 0)),
        ],
        out_shape=[
            jax.ShapeDtypeStruct((NPAD, 16), F32),
            jax.ShapeDtypeStruct((NPAD, 16), F32),
        ],
    )(acc1, acc1, rmat, wt, wd2)


def _k5_body(a0_ref, a1_ref, o_ref):
    s = a0_ref[...] + a1_ref[...]
    numer = s[:, 0:OUT]
    den = s[:, OUT:OUT + 1]
    o_ref[...] = jnp.maximum(numer, 0.0) / (den + 1e-9)


def _k5(acc2):
    return pl.pallas_call(
        _k5_body,
        grid=(NBLK2,),
        in_specs=[
            pl.BlockSpec((BLK2, 16), lambda i: (i, 0)),
            pl.BlockSpec((BLK2, 16), lambda i: (i + NBLK2, 0)),
        ],
        out_specs=pl.BlockSpec((BLK2, OUT), lambda i: (i, 0)),
        out_shape=jax.ShapeDtypeStruct((NPAD, OUT), F32),
    )(acc2, acc2)


def _sc_mesh():
    return plsc.VectorSubcoreMesh(core_axis_name="c", subcore_axis_name="s")


def _zero_acc(zbuf, acc, width, s):
    zero = jnp.zeros((L,), F32)
    nvec = width // L

    def zrow(i, _):
        for j in range(nvec):
            zbuf[i, pl.ds(j * L, L)] = zero
        return 0

    lax.fori_loop(0, 128, zrow, 0)
    for k in range(5):
        pltpu.sync_copy(zbuf, acc.at[pl.ds(s * RPT + k * 128, 128)])


def _sc_layer1(tsrc, tdst, src_ix, dst_ix):
    return _sc_edge_kernel(tsrc, tdst, src_ix, dst_ix, 80, _edges1)


def _edges1(srows, drows, vals):
    giota = lax.iota(jnp.int32, L)
    mask8 = giota < 8
    half = jnp.where(mask8, 0, 1)

    @plsc.parallel_loop(0, CH, unroll=8)
    def edge(e):
        t = srows[e, pl.ds(64, L)] + drows[e, :]
        t = jnp.maximum(t, 0.2 * t)
        sc = jnp.exp(t)
        vals[e, pl.ds(64, L)] = jnp.where(mask8, sc, 0.0)
        for j in range(4):
            b = jnp.take(sc, 2 * j + half)
            vals[e, pl.ds(16 * j, L)] = b * srows[e, pl.ds(16 * j, L)]


def _sc_edge_kernel(tsrc, tdst, src_ix, dst_ix, width, edge_fn):
    @functools.partial(
        pl.kernel,
        out_type=jax.ShapeDtypeStruct((2 * NPAD, width), F32),
        mesh=_sc_mesh(),
        scratch_types=[
            pltpu.VMEM((EW,), jnp.int32),
            pltpu.VMEM((EW,), jnp.int32),
            pltpu.VMEM((2, CH, width), F32),
            pltpu.VMEM((2, CH, 16), F32),
            pltpu.VMEM((2, CH, width), F32),
            pltpu.VMEM((CH,), jnp.int32),
            pltpu.VMEM((CH,), jnp.int32),
            pltpu.VMEM((128, width), F32),
            pltpu.VMEM_SHARED((NPAD, width), F32),
            pltpu.SemaphoreType.DMA,
            pltpu.SemaphoreType.DMA,
            pltpu.SemaphoreType.DMA,
            pltpu.SemaphoreType.DMA,
        ],
        compiler_params=pltpu.CompilerParams(use_tc_tiling_on_sc=False),
    )
    def k(tsrc_h, tdst_h, six_h, dix_h, out_h,
          sixall, dixall, srows2, drows2, vals2, sdix0, sdix1, zbuf, acc,
          gsem0, gsem1, ssem0, ssem1):
        c = lax.axis_index("c")
        s = lax.axis_index("s")
        w = s * NC + c

        _zero_acc(zbuf, acc, width, s)

        ebase = pl.multiple_of(w * EW, 8)
        pltpu.sync_copy(six_h.at[pl.ds(ebase, EW)], sixall)
        pltpu.sync_copy(dix_h.at[pl.ds(ebase, EW)], dixall)
        plsc.subcore_barrier()

        bufs = [
            (srows2.at[0], drows2.at[0], vals2.at[0], sdix0, gsem0, ssem0),
            (srows2.at[1], drows2.at[1], vals2.at[1], sdix1, gsem1, ssem1),
        ]

        H = CH // 2

        def start_gather(ci, b):
            srows, drows, _, _, gsem, _ = bufs[b]
            off = ci * CH
            pltpu.async_copy(tsrc_h.at[sixall.at[pl.ds(off, H)]],
                             srows.at[pl.ds(0, H)], gsem)
            pltpu.async_copy(tsrc_h.at[sixall.at[pl.ds(off + H, H)]],
                             srows.at[pl.ds(H, H)], gsem)
            pltpu.async_copy(tdst_h.at[dixall.at[pl.ds(off, CH)]], drows, gsem)

        def wait_gather(b):
            srows, drows, _, _, gsem, _ = bufs[b]
            pltpu.make_async_copy(tsrc_h.at[sixall.at[pl.ds(0, CH)]], srows, gsem).wait()
            pltpu.make_async_copy(tdst_h.at[dixall.at[pl.ds(0, CH)]], drows, gsem).wait()

        def wait_scatter(b):
            _, _, vals, sdix, _, ssem = bufs[b]
            pltpu.make_async_copy(vals, acc.at[sdix], ssem).wait()

        def process(ci, b, first):
            srows, drows, vals, sdix, _, ssem = bufs[b]
            wait_gather(b)
            if first:
                pl.when(ci >= 2)(lambda: wait_scatter(b))
            else:
                wait_scatter(b)
            off = ci * CH
            for j in range(CH // L):
                sdix[pl.ds(j * L, L)] = dixall[pl.ds(off + j * L, L)]
            edge_fn(srows, drows, vals)
            pltpu.async_copy(vals, acc.at[sdix], ssem, add=True)

        start_gather(0, 0)

        def pair(i, _):
            start_gather(2 * i + 1, 1)
            process(2 * i, 0, True)
            start_gather(2 * i + 2, 0)
            process(2 * i + 1, 1, True)
            return 0

        lax.fori_loop(0, (NCHUNK - 1) // 2, pair, 0)
        process(NCHUNK - 1, 0, False)
        wait_scatter(1)
        wait_scatter(0)

        plsc.subcore_barrier()
        pltpu.sync_copy(acc.at[pl.ds(s * RPT, RPT)],
                        out_h.at[pl.ds(c * NPAD + s * RPT, RPT)])

    return k(tsrc, tdst, src_ix, dst_ix)


def _sc_layer2(tsrc, sd1d, ss1d, src_ix, dst_ix):
    @functools.partial(
        pl.kernel,
        out_type=jax.ShapeDtypeStruct((2 * NPAD, 16), F32),
        mesh=_sc_mesh(),
        scratch_types=[
            pltpu.VMEM((EW,), jnp.int32),
            pltpu.VMEM((EW,), jnp.int32),
            pltpu.VMEM((2, CH, 16), F32),
            pltpu.VMEM((2, CH, 16), F32),
            pltpu.VMEM((CH,), jnp.int32),
            pltpu.VMEM((CH,), jnp.int32),
            pltpu.VMEM((2, CH), F32),
            pltpu.VMEM((2, CH), F32),
            pltpu.VMEM((128, 16), F32),
            pltpu.VMEM_SHARED((NPAD, 16), F32),
            pltpu.SemaphoreType.DMA,
            pltpu.SemaphoreType.DMA,
            pltpu.SemaphoreType.DMA,
            pltpu.SemaphoreType.DMA,
        ],
        compiler_params=pltpu.CompilerParams(use_tc_tiling_on_sc=False),
    )
    def k(tsrc_h, sd_h, ss_h, six_h, dix_h, out_h,
          sixall, dixall, srows2, vals2, sdix0, sdix1, sdch2, ssch2, zbuf,
          acc, gsem0, gsem1, ssem0, ssem1):
        c = lax.axis_index("c")
        s = lax.axis_index("s")
        w = s * NC + c

        _zero_acc(zbuf, acc, 16, s)

        ebase = pl.multiple_of(w * EW, 8)
        pltpu.sync_copy(six_h.at[pl.ds(ebase, EW)], sixall)
        pltpu.sync_copy(dix_h.at[pl.ds(ebase, EW)], dixall)
        plsc.subcore_barrier()

        giota = lax.iota(jnp.int32, L)
        mask8 = giota < 8
        lane_consts = [jnp.full((L,), e, jnp.int32) for e in range(L)]
        bufs = [
            (srows2.at[0], vals2.at[0], sdix0, sdch2.at[0], ssch2.at[0], gsem0, ssem0),
            (srows2.at[1], vals2.at[1], sdix1, sdch2.at[1], ssch2.at[1], gsem1, ssem1),
        ]

        H = CH // 2

        def start_gather(ci, b):
            srows, _, _, sdch, ssch, gsem, _ = bufs[b]
            off = ci * CH
            pltpu.async_copy(tsrc_h.at[sixall.at[pl.ds(off, H)]],
                             srows.at[pl.ds(0, H)], gsem)
            pltpu.async_copy(tsrc_h.at[sixall.at[pl.ds(off + H, H)]],
                             srows.at[pl.ds(H, H)], gsem)
            pltpu.async_copy(sd_h.at[dixall.at[pl.ds(off, CH)]], sdch, gsem)
            pltpu.async_copy(ss_h.at[sixall.at[pl.ds(off, CH)]], ssch, gsem)

        def wait_gather(b):
            srows, _, _, sdch, ssch, gsem, _ = bufs[b]
            pltpu.make_async_copy(tsrc_h.at[sixall.at[pl.ds(0, CH)]], srows, gsem).wait()
            pltpu.make_async_copy(sd_h.at[dixall.at[pl.ds(0, CH)]], sdch, gsem).wait()
            pltpu.make_async_copy(ss_h.at[sixall.at[pl.ds(0, CH)]], ssch, gsem).wait()

        def wait_scatter(b):
            _, vals, sdix, _, _, _, ssem = bufs[b]
            pltpu.make_async_copy(vals, acc.at[sdix], ssem).wait()

        def process(ci, b, guard):
            srows, vals, sdix, sdch, ssch, _, ssem = bufs[b]
            wait_gather(b)
            if guard:
                pl.when(ci >= 2)(lambda: wait_scatter(b))
            else:
                wait_scatter(b)
            off = ci * CH
            for j in range(CH // L):
                sdix[pl.ds(j * L, L)] = dixall[pl.ds(off + j * L, L)]
            @plsc.parallel_loop(0, CH // L, unroll=5)
            def group(kk):
                t = sdch[pl.ds(kk * L, L)] + ssch[pl.ds(kk * L, L)]
                t = jnp.maximum(t, 0.2 * t)
                sc16 = jnp.exp(t)
                for e in range(L):
                    b_ = jnp.take(sc16, lane_consts[e])
                    vals[kk * L + e, :] = jnp.where(mask8, b_ * srows[kk * L + e, :], 0.0)
            pltpu.async_copy(vals, acc.at[sdix], ssem, add=True)

        start_gather(0, 0)

        def pair(i, _):
            start_gather(2 * i + 1, 1)
            process(2 * i, 0, True)
            start_gather(2 * i + 2, 0)
            process(2 * i + 1, 1, True)
            return 0

        lax.fori_loop(0, (NCHUNK - 1) // 2, pair, 0)
        process(NCHUNK - 1, 0, False)
        wait_scatter(1)
        wait_scatter(0)

        plsc.subcore_barrier()
        pltpu.sync_copy(acc.at[pl.ds(s * RPT, RPT)],
                        out_h.at[pl.ds(c * NPAD + s * RPT, RPT)])

    return k(tsrc, sd1d, ss1d, src_ix, dst_ix)


def kernel(node_states, edges, W1, A1, W2, A2, training=False):
    del training
    x = node_states.astype(F32)

    wc = W1.transpose(1, 0, 2).reshape(F, H1 * U1)
    eye = jnp.eye(H1, dtype=F32)
    a_dst = A1[:, :U1, 0]
    a_src = A1[:, U1:, 0]
    ad_mat = (a_dst[:, :, None] * eye[:, None, :]).reshape(H1 * U1, H1)
    as_mat = (a_src[:, :, None] * eye[:, None, :]).reshape(H1 * U1, H1)
    z8 = jnp.zeros((F, 8), F32)
    wsrc = jnp.concatenate([wc, wc @ as_mat, z8], axis=1)
    wdst = jnp.concatenate([wc @ ad_mat, z8], axis=1)

    rmat = (eye[:, :, None] * jnp.ones((1, 1, U1), F32)).reshape(H1, H1 * U1)

    w2 = W2[0]
    a2d = A2[0, :OUT, 0]
    a2s = A2[0, OUT:, 0]
    z1 = jnp.zeros((H1 * U1, 1), F32)
    z7 = jnp.zeros((H1 * U1, 7), F32)
    wt = jnp.concatenate([w2, z1, (w2 @ a2s)[:, None], z7], axis=1)
    wd2 = jnp.concatenate([jnp.zeros((H1 * U1, 8), F32),
                           (w2 @ a2d)[:, None], z7], axis=1)

    src = edges[:, 1].astype(jnp.int32)
    dst = edges[:, 0].astype(jnp.int32)

    tsrc1, tdst1 = _k1(x, wsrc, wdst)
    acc1 = _sc_layer1(tsrc1, tdst1, src, dst)
    tsrc2, tdst2 = _k3(acc1, rmat, wt, wd2)
    sd1d = tdst2[:, 8]
    ss1d = tsrc2[:, 8]
    acc2 = _sc_layer2(tsrc2, sd1d, ss1d, src, dst)
    return _k5(acc2)[:N]


# speedup vs baseline: 1.0473x; 1.0472x over previous
import functools
import jax
import jax.numpy as jnp
from jax import lax
from jax.experimental import pallas as pl
from jax.experimental.pallas import tpu as pltpu
from jax.experimental.pallas import tpu_sc as plsc

F32 = jnp.float32
HIGH = lax.Precision.HIGHEST

N = 10000
E = 320000
F = 128
H1, U1 = 8, 8
OUT = 7

NC, NS, L = 2, 16, 16
NW = NC * NS
EW = E // NW
CH = 128
NCHUNK = EW // CH
TAIL = EW - NCHUNK * CH
NPAD = 10240
RPT = NPAD // NS

BLK = 400
NBLK = N // BLK
BLK2 = 512
NBLK2 = NPAD // BLK2


def _dot(a, b):
    return jnp.dot(a, b, preferred_element_type=F32, precision=HIGH)


def _k1_body(x_ref, ws_ref, wd_ref, ts_ref, td_ref):
    x = x_ref[...]
    ts_ref[...] = _dot(x, ws_ref[...])
    td_ref[...] = _dot(x, wd_ref[...])


def _k1(x, wsrc, wdst):
    return pl.pallas_call(
        _k1_body,
        grid=(NBLK,),
        in_specs=[
            pl.BlockSpec((BLK, F), lambda i: (i, 0)),
            pl.BlockSpec((F, 80), lambda i: (0, 0)),
            pl.BlockSpec((F, 16), lambda i: (0, 0)),
        ],
        out_specs=[
            pl.BlockSpec((BLK, 80), lambda i: (i, 0)),
            pl.BlockSpec((BLK, 16), lambda i: (i, 0)),
        ],
        out_shape=[
            jax.ShapeDtypeStruct((N, 80), F32),
            jax.ShapeDtypeStruct((N, 16), F32),
        ],
    )(x, wsrc, wdst)


def _k3_body(a0_ref, a1_ref, r_ref, wt_ref, wd_ref, ts_ref, td_ref):
    s = a0_ref[...] + a1_ref[...]
    numer = s[:, 0:64]
    den = s[:, 64:72]
    rrep = _dot(1.0 / (den + 1e-9), r_ref[...])
    x2 = jnp.maximum(numer, 0.0) * rrep
    t = _dot(x2, wt_ref[...])
    col = lax.broadcasted_iota(jnp.int32, t.shape, 1)
    ts_ref[...] = jnp.where(col == 7, 1.0, t)
    td_ref[...] = _dot(x2, wd_ref[...])


def _k3(acc1, rmat, wt, wd2):
    return pl.pallas_call(
        _k3_body,
        grid=(NBLK2,),
        in_specs=[
            pl.BlockSpec((BLK2, 80), lambda i: (i, 0)),
            pl.BlockSpec((BLK2, 80), lambda i: (i + NBLK2, 0)),
            pl.BlockSpec((H1, 64), lambda i: (0, 0)),
            pl.BlockSpec((64, 16), lambda i: (0, 0)),
            pl.BlockSpec((64, 16), lambda i: (0, 0)),
        ],
        out_specs=[
            pl.BlockSpec((BLK2, 16), lambda i: (i, 0)),
            pl.BlockSpec((BLK2, 16), lambda i: (i, 0)),
        ],
        out_shape=[
            jax.ShapeDtypeStruct((NPAD, 16), F32),
            jax.ShapeDtypeStruct((NPAD, 16), F32),
        ],
    )(acc1, acc1, rmat, wt, wd2)


def _k5_body(a0_ref, a1_ref, o_ref):
    s = a0_ref[...] + a1_ref[...]
    numer = s[:, 0:OUT]
    den = s[:, OUT:OUT + 1]
    o_ref[...] = jnp.maximum(numer, 0.0) / (den + 1e-9)


def _k5(acc2):
    return pl.pallas_call(
        _k5_body,
        grid=(NBLK2,),
        in_specs=[
            pl.BlockSpec((BLK2, 16), lambda i: (i, 0)),
            pl.BlockSpec((BLK2, 16), lambda i: (i + NBLK2, 0)),
        ],
        out_specs=pl.BlockSpec((BLK2, OUT), lambda i: (i, 0)),
        out_shape=jax.ShapeDtypeStruct((NPAD, OUT), F32),
    )(acc2, acc2)


def _sc_mesh():
    return plsc.VectorSubcoreMesh(core_axis_name="c", subcore_axis_name="s")


def _zero_acc(zbuf, acc, width, s):
    zero = jnp.zeros((L,), F32)
    nvec = width // L

    def zrow(i, _):
        for j in range(nvec):
            zbuf[i, pl.ds(j * L, L)] = zero
        return 0

    lax.fori_loop(0, 128, zrow, 0)
    for k in range(5):
        pltpu.sync_copy(zbuf, acc.at[pl.ds(s * RPT + k * 128, 128)])


def _sc_layer1(tsrc, tdst, src_ix, dst_ix):
    return _sc_edge_kernel(tsrc, tdst, src_ix, dst_ix, 80, _edges1)


def _edges1(srows, drows, vals, ch):
    giota = lax.iota(jnp.int32, L)
    mask8 = giota < 8
    half = jnp.where(mask8, 0, 1)

    @plsc.parallel_loop(0, ch, unroll=8)
    def edge(e):
        t = srows[e, pl.ds(64, L)] + drows[e, :]
        t = jnp.maximum(t, 0.2 * t)
        sc = jnp.exp(t)
        vals[e, pl.ds(64, L)] = jnp.where(mask8, sc, 0.0)
        for j in range(4):
            b = jnp.take(sc, 2 * j + half)
            vals[e, pl.ds(16 * j, L)] = b * srows[e, pl.ds(16 * j, L)]


def _sc_edge_kernel(tsrc, tdst, src_ix, dst_ix, width, edge_fn):
    @functools.partial(
        pl.kernel,
        out_type=jax.ShapeDtypeStruct((2 * NPAD, width), F32),
        mesh=_sc_mesh(),
        scratch_types=[
            pltpu.VMEM((EW,), jnp.int32),
            pltpu.VMEM((EW,), jnp.int32),
            pltpu.VMEM((2, CH, width), F32),
            pltpu.VMEM((2, CH, 16), F32),
            pltpu.VMEM((2, CH, width), F32),
            pltpu.VMEM((CH,), jnp.int32),
            pltpu.VMEM((CH,), jnp.int32),
            pltpu.VMEM((TAIL,), jnp.int32),
            pltpu.VMEM((128, width), F32),
            pltpu.VMEM_SHARED((NPAD, width), F32),
            pltpu.SemaphoreType.DMA,
            pltpu.SemaphoreType.DMA,
            pltpu.SemaphoreType.DMA,
            pltpu.SemaphoreType.DMA,
        ],
        compiler_params=pltpu.CompilerParams(use_tc_tiling_on_sc=False),
    )
    def k(tsrc_h, tdst_h, six_h, dix_h, out_h,
          sixall, dixall, srows2, drows2, vals2, sdix0, sdix1, sdixt, zbuf, acc,
          gsem0, gsem1, ssem0, ssem1):
        c = lax.axis_index("c")
        s = lax.axis_index("s")
        w = s * NC + c

        _zero_acc(zbuf, acc, width, s)

        ebase = pl.multiple_of(w * EW, 8)
        pltpu.sync_copy(six_h.at[pl.ds(ebase, EW)], sixall)
        pltpu.sync_copy(dix_h.at[pl.ds(ebase, EW)], dixall)
        plsc.subcore_barrier()

        bufs = [
            (srows2.at[0], drows2.at[0], vals2.at[0], sdix0, gsem0, ssem0),
            (srows2.at[1], drows2.at[1], vals2.at[1], sdix1, gsem1, ssem1),
        ]

        def start_gather(ci, b, ch=CH):
            srows, drows, _, _, gsem, _ = bufs[b]
            off = ci * CH
            pltpu.async_copy(tsrc_h.at[sixall.at[pl.ds(off, ch)]],
                             srows.at[pl.ds(0, ch)], gsem)
            pltpu.async_copy(tdst_h.at[dixall.at[pl.ds(off, ch)]],
                             drows.at[pl.ds(0, ch)], gsem)

        def wait_gather(b, ch=CH):
            srows, drows, _, _, gsem, _ = bufs[b]
            pltpu.make_async_copy(tsrc_h.at[sixall.at[pl.ds(0, ch)]],
                                  srows.at[pl.ds(0, ch)], gsem).wait()
            pltpu.make_async_copy(tdst_h.at[dixall.at[pl.ds(0, ch)]],
                                  drows.at[pl.ds(0, ch)], gsem).wait()

        def wait_scatter(b):
            _, _, vals, sdix, _, ssem = bufs[b]
            pltpu.make_async_copy(vals, acc.at[sdix], ssem).wait()

        def process(ci, b, guard):
            srows, drows, vals, sdix, _, ssem = bufs[b]
            wait_gather(b)
            if guard:
                pl.when(ci >= 2)(lambda: wait_scatter(b))
            else:
                wait_scatter(b)
            off = ci * CH
            for j in range(CH // L):
                sdix[pl.ds(j * L, L)] = dixall[pl.ds(off + j * L, L)]
            edge_fn(srows, drows, vals, CH)
            pltpu.async_copy(vals, acc.at[sdix], ssem, add=True)

        start_gather(0, 0)

        def pair(i, _):
            start_gather(2 * i + 1, 1)
            process(2 * i, 0, True)
            start_gather(2 * i + 2, 0)
            process(2 * i + 1, 1, True)
            return 0

        lax.fori_loop(0, (NCHUNK - 2) // 2, pair, 0)
        start_gather(NCHUNK - 1, 1)
        process(NCHUNK - 2, 0, False)
        process(NCHUNK - 1, 1, False)

        srows0, drows0, vals0, _, _, ssem0_ = bufs[0]
        start_gather(NCHUNK, 0, TAIL)
        wait_gather(0, TAIL)
        wait_scatter(0)
        off_t = NCHUNK * CH
        sdixt[pl.ds(0, L)] = dixall[pl.ds(off_t, L)]
        edge_fn(srows0, drows0, vals0, TAIL)
        pltpu.async_copy(vals0.at[pl.ds(0, TAIL)], acc.at[sdixt], ssem0_, add=True)
        pltpu.make_async_copy(vals0.at[pl.ds(0, TAIL)], acc.at[sdixt], ssem0_).wait()
        wait_scatter(1)

        plsc.subcore_barrier()
        pltpu.sync_copy(acc.at[pl.ds(s * RPT, RPT)],
                        out_h.at[pl.ds(c * NPAD + s * RPT, RPT)])

    return k(tsrc, tdst, src_ix, dst_ix)


def _sc_layer2(tsrc, sd1d, ss1d, src_ix, dst_ix):
    @functools.partial(
        pl.kernel,
        out_type=jax.ShapeDtypeStruct((2 * NPAD, 16), F32),
        mesh=_sc_mesh(),
        scratch_types=[
            pltpu.VMEM((EW,), jnp.int32),
            pltpu.VMEM((EW,), jnp.int32),
            pltpu.VMEM((2, CH, 16), F32),
            pltpu.VMEM((2, CH, 16), F32),
            pltpu.VMEM((CH,), jnp.int32),
            pltpu.VMEM((CH,), jnp.int32),
            pltpu.VMEM((TAIL,), jnp.int32),
            pltpu.VMEM((2, CH), F32),
            pltpu.VMEM((2, CH), F32),
            pltpu.VMEM((128, 16), F32),
            pltpu.VMEM_SHARED((NPAD, 16), F32),
            pltpu.SemaphoreType.DMA,
            pltpu.SemaphoreType.DMA,
            pltpu.SemaphoreType.DMA,
            pltpu.SemaphoreType.DMA,
        ],
        compiler_params=pltpu.CompilerParams(use_tc_tiling_on_sc=False),
    )
    def k(tsrc_h, sd_h, ss_h, six_h, dix_h, out_h,
          sixall, dixall, srows2, vals2, sdix0, sdix1, sdixt, sdch2, ssch2, zbuf,
          acc, gsem0, gsem1, ssem0, ssem1):
        c = lax.axis_index("c")
        s = lax.axis_index("s")
        w = s * NC + c

        _zero_acc(zbuf, acc, 16, s)

        ebase = pl.multiple_of(w * EW, 8)
        pltpu.sync_copy(six_h.at[pl.ds(ebase, EW)], sixall)
        pltpu.sync_copy(dix_h.at[pl.ds(ebase, EW)], dixall)
        plsc.subcore_barrier()

        giota = lax.iota(jnp.int32, L)
        mask8 = giota < 8
        lane_consts = [jnp.full((L,), e, jnp.int32) for e in range(L)]
        bufs = [
            (srows2.at[0], vals2.at[0], sdix0, sdch2.at[0], ssch2.at[0], gsem0, ssem0),
            (srows2.at[1], vals2.at[1], sdix1, sdch2.at[1], ssch2.at[1], gsem1, ssem1),
        ]

        def start_gather(ci, b, ch=CH):
            srows, _, _, sdch, ssch, gsem, _ = bufs[b]
            off = ci * CH
            pltpu.async_copy(tsrc_h.at[sixall.at[pl.ds(off, ch)]],
                             srows.at[pl.ds(0, ch)], gsem)
            pltpu.async_copy(sd_h.at[dixall.at[pl.ds(off, ch)]],
                             sdch.at[pl.ds(0, ch)], gsem)
            pltpu.async_copy(ss_h.at[sixall.at[pl.ds(off, ch)]],
                             ssch.at[pl.ds(0, ch)], gsem)

        def wait_gather(b, ch=CH):
            srows, _, _, sdch, ssch, gsem, _ = bufs[b]
            pltpu.make_async_copy(tsrc_h.at[sixall.at[pl.ds(0, ch)]],
                                  srows.at[pl.ds(0, ch)], gsem).wait()
            pltpu.make_async_copy(sd_h.at[dixall.at[pl.ds(0, ch)]],
                                  sdch.at[pl.ds(0, ch)], gsem).wait()
            pltpu.make_async_copy(ss_h.at[sixall.at[pl.ds(0, ch)]],
                                  ssch.at[pl.ds(0, ch)], gsem).wait()

        def wait_scatter(b):
            _, vals, sdix, _, _, _, ssem = bufs[b]
            pltpu.make_async_copy(vals, acc.at[sdix], ssem).wait()

        def compute(b, ngrp):
            srows, vals, _, sdch, ssch, _, _ = bufs[b]

            @plsc.parallel_loop(0, ngrp, unroll=min(ngrp, 4))
            def group(kk):
                t = sdch[pl.ds(kk * L, L)] + ssch[pl.ds(kk * L, L)]
                t = jnp.maximum(t, 0.2 * t)
                sc16 = jnp.exp(t)
                for e in range(L):
                    b_ = jnp.take(sc16, lane_consts[e])
                    vals[kk * L + e, :] = jnp.where(mask8, b_ * srows[kk * L + e, :], 0.0)

        def process(ci, b, guard):
            srows, vals, sdix, sdch, ssch, _, ssem = bufs[b]
            wait_gather(b)
            if guard:
                pl.when(ci >= 2)(lambda: wait_scatter(b))
            else:
                wait_scatter(b)
            off = ci * CH
            for j in range(CH // L):
                sdix[pl.ds(j * L, L)] = dixall[pl.ds(off + j * L, L)]
            compute(b, CH // L)
            pltpu.async_copy(vals, acc.at[sdix], ssem, add=True)

        start_gather(0, 0)

        def pair(i, _):
            start_gather(2 * i + 1, 1)
            process(2 * i, 0, True)
            start_gather(2 * i + 2, 0)
            process(2 * i + 1, 1, True)
            return 0

        lax.fori_loop(0, (NCHUNK - 2) // 2, pair, 0)
        start_gather(NCHUNK - 1, 1)
        process(NCHUNK - 2, 0, False)
        process(NCHUNK - 1, 1, False)

        _, vals0, _, _, _, _, ssem0_ = bufs[0]
        start_gather(NCHUNK, 0, TAIL)
        wait_gather(0, TAIL)
        wait_scatter(0)
        off_t = NCHUNK * CH
        sdixt[pl.ds(0, L)] = dixall[pl.ds(off_t, L)]
        compute(0, TAIL // L)
        pltpu.async_copy(vals0.at[pl.ds(0, TAIL)], acc.at[sdixt], ssem0_, add=True)
        pltpu.make_async_copy(vals0.at[pl.ds(0, TAIL)], acc.at[sdixt], ssem0_).wait()
        wait_scatter(1)

        plsc.subcore_barrier()
        pltpu.sync_copy(acc.at[pl.ds(s * RPT, RPT)],
                        out_h.at[pl.ds(c * NPAD + s * RPT, RPT)])

    return k(tsrc, sd1d, ss1d, src_ix, dst_ix)


def kernel(node_states, edges, W1, A1, W2, A2, training=False):
    del training
    x = node_states.astype(F32)

    wc = W1.transpose(1, 0, 2).reshape(F, H1 * U1)
    eye = jnp.eye(H1, dtype=F32)
    a_dst = A1[:, :U1, 0]
    a_src = A1[:, U1:, 0]
    ad_mat = (a_dst[:, :, None] * eye[:, None, :]).reshape(H1 * U1, H1)
    as_mat = (a_src[:, :, None] * eye[:, None, :]).reshape(H1 * U1, H1)
    z8 = jnp.zeros((F, 8), F32)
    wsrc = jnp.concatenate([wc, wc @ as_mat, z8], axis=1)
    wdst = jnp.concatenate([wc @ ad_mat, z8], axis=1)

    rmat = (eye[:, :, None] * jnp.ones((1, 1, U1), F32)).reshape(H1, H1 * U1)

    w2 = W2[0]
    a2d = A2[0, :OUT, 0]
    a2s = A2[0, OUT:, 0]
    z1 = jnp.zeros((H1 * U1, 1), F32)
    z7 = jnp.zeros((H1 * U1, 7), F32)
    wt = jnp.concatenate([w2, z1, (w2 @ a2s)[:, None], z7], axis=1)
    wd2 = jnp.concatenate([jnp.zeros((H1 * U1, 8), F32),
                           (w2 @ a2d)[:, None], z7], axis=1)

    src = edges[:, 1].astype(jnp.int32)
    dst = edges[:, 0].astype(jnp.int32)

    tsrc1, tdst1 = _k1(x, wsrc, wdst)
    acc1 = _sc_layer1(tsrc1, tdst1, src, dst)
    tsrc2, tdst2 = _k3(acc1, rmat, wt, wd2)
    sd1d = tdst2[:, 8]
    ss1d = tsrc2[:, 8]
    acc2 = _sc_layer2(tsrc2, sd1d, ss1d, src, dst)
    return _k5(acc2)[:N]
